# speedup vs baseline: 1.0274x; 1.0274x over previous
import jax
import jax.numpy as jnp
from jax.experimental import pallas as pl

B = 2
N = 100000
FEAT = 64
GRID = 512
XMIN = -51.2
VS = 0.2


def _finalize_body(src_ref, tgt_ref, cs_ref, ct_ref, tf_ref, o_ref):
    cs = cs_ref[...]
    ct = ct_ref[...]
    occ = ((cs + ct) > 0.0).astype(jnp.float32)
    inv_s = 1.0 / jnp.maximum(cs, 1.0)
    inv_t = 1.0 / jnp.maximum(ct, 1.0)
    tf = tf_ref[...]
    o_ref[...] = (tgt_ref[...] * inv_t - src_ref[...] * inv_s + tf) * occ


def kernel(pc0s, pc1s, W_feat, b_feat, W_time, b_time, time_idx):
    nseg = B * GRID * GRID

    def voxelize(pc):
        vx = jnp.clip(jnp.floor((pc[..., 0] - XMIN) / VS), 0, GRID - 1).astype(jnp.int32)
        vy = jnp.clip(jnp.floor((pc[..., 1] - XMIN) / VS), 0, GRID - 1).astype(jnp.int32)
        bid = jnp.arange(B, dtype=jnp.int32)[:, None]
        seg = (bid * GRID * GRID + vx * GRID + vy).reshape(-1)
        feats = jax.nn.relu(pc.reshape(-1, 3) @ W_feat + b_feat)
        sums = jax.ops.segment_sum(feats, seg, num_segments=nseg)
        cnts = jax.ops.segment_sum(jnp.ones((B * N,), jnp.float32), seg, num_segments=nseg)
        return sums, cnts

    src_sums, src_cnt = voxelize(pc0s)
    tgt_sums, tgt_cnt = voxelize(pc1s)
    time_feat = W_time[time_idx] + b_time

    ROWS = 4096
    grid = (nseg // ROWS,)
    out = pl.pallas_call(
        _finalize_body,
        grid=grid,
        in_specs=[
            pl.BlockSpec((ROWS, FEAT), lambda i: (i, 0)),
            pl.BlockSpec((ROWS, FEAT), lambda i: (i, 0)),
            pl.BlockSpec((ROWS, 1), lambda i: (i, 0)),
            pl.BlockSpec((ROWS, 1), lambda i: (i, 0)),
            pl.BlockSpec((1, FEAT), lambda i: (0, 0)),
        ],
        out_specs=pl.BlockSpec((ROWS, FEAT), lambda i: (i, 0)),
        out_shape=jax.ShapeDtypeStruct((nseg, FEAT), jnp.float32),
    )(src_sums, tgt_sums, src_cnt[:, None], tgt_cnt[:, None], time_feat[None, :])
    return out.reshape(B, GRID, GRID, FEAT)


# speedup vs baseline: 1.0389x; 1.0112x over previous
import jax
import jax.numpy as jnp
from jax.experimental import pallas as pl

B = 2
N = 100000
FEAT = 64
GRID = 512
XMIN = -51.2
VS = 0.2


def _finalize_body(src_ref, tgt_ref, cs_ref, ct_ref, tf_ref, o_ref):
    cs = cs_ref[...]
    ct = ct_ref[...]
    occ = ((cs + ct) > 0.0).astype(jnp.float32)
    inv_s = 1.0 / jnp.maximum(cs, 1.0)
    inv_t = 1.0 / jnp.maximum(ct, 1.0)
    tf = tf_ref[...]
    o_ref[...] = (tgt_ref[...] * inv_t - src_ref[...] * inv_s + tf) * occ


def kernel(pc0s, pc1s, W_feat, b_feat, W_time, b_time, time_idx):
    nseg = B * GRID * GRID

    def voxelize(pc):
        vx = jnp.clip(jnp.floor((pc[..., 0] - XMIN) / VS), 0, GRID - 1).astype(jnp.int32)
        vy = jnp.clip(jnp.floor((pc[..., 1] - XMIN) / VS), 0, GRID - 1).astype(jnp.int32)
        bid = jnp.arange(B, dtype=jnp.int32)[:, None]
        seg = (bid * GRID * GRID + vx * GRID + vy).reshape(-1)
        feats = jax.nn.relu(pc.reshape(-1, 3) @ W_feat + b_feat)
        ext = jnp.concatenate([feats, jnp.ones((B * N, 1), jnp.float32)], axis=1)
        sums = jax.ops.segment_sum(ext, seg, num_segments=nseg)
        return sums[:, :FEAT], sums[:, FEAT]

    src_sums, src_cnt = voxelize(pc0s)
    tgt_sums, tgt_cnt = voxelize(pc1s)
    time_feat = W_time[time_idx] + b_time

    ROWS = 4096
    grid = (nseg // ROWS,)
    out = pl.pallas_call(
        _finalize_body,
        grid=grid,
        in_specs=[
            pl.BlockSpec((ROWS, FEAT), lambda i: (i, 0)),
            pl.BlockSpec((ROWS, FEAT), lambda i: (i, 0)),
            pl.BlockSpec((ROWS, 1), lambda i: (i, 0)),
            pl.BlockSpec((ROWS, 1), lambda i: (i, 0)),
            pl.BlockSpec((1, FEAT), lambda i: (0, 0)),
        ],
        out_specs=pl.BlockSpec((ROWS, FEAT), lambda i: (i, 0)),
        out_shape=jax.ShapeDtypeStruct((nseg, FEAT), jnp.float32),
    )(src_sums, tgt_sums, src_cnt[:, None], tgt_cnt[:, None], time_feat[None, :])
    return out.reshape(B, GRID, GRID, FEAT)
